# baseline (speedup 1.0000x reference)
import functools
import jax
import jax.numpy as jnp
from jax import lax
from jax.experimental import pallas as pl
from jax.experimental.pallas import tpu as pltpu
from jax.experimental.pallas import tpu_sc as plsc

_N = 10000
_D = 128
_E = 320000
_TILES = 16
_PASSES = 3
_OWN = 240
_LACC = 256
_SPAN = _TILES * _OWN
_SIDE = _PASSES * _SPAN
_GEDGES = 2048
_EPAD = 321536
_NGRP = _EPAD // _GEDGES
_BLK = 64
_PCAP = _GEDGES + 2 * _BLK


def _sc_segment_sums(x_cat, r_all, c_all):
    mesh = plsc.VectorSubcoreMesh(core_axis_name="c", subcore_axis_name="s")

    @functools.partial(
        pl.kernel,
        out_type=[
            jax.ShapeDtypeStruct((2 * _SIDE, _D), jnp.float32),
            jax.ShapeDtypeStruct((2 * _SIDE, 16), jnp.float32),
        ],
        mesh=mesh,
        compiler_params=pltpu.CompilerParams(needs_layout_passes=False),
        scratch_types=[
            pltpu.VMEM((2 * _GEDGES,), jnp.int32),
            pltpu.VMEM((2 * _GEDGES,), jnp.int32),
            pltpu.VMEM((_PCAP,), jnp.int32),
            pltpu.VMEM((_PCAP,), jnp.int32),
            pltpu.VMEM((2 * _BLK, _D), jnp.float32),
            pltpu.VMEM((_LACC, _D), jnp.float32),
            pltpu.VMEM((_LACC, 16), jnp.float32),
            pltpu.SemaphoreType.DMA,
            pltpu.SemaphoreType.DMA,
        ],
    )
    def sc_fn(x_hbm, r_hbm, c_hbm, zacc_hbm, zcnt_hbm,
              acc_out, cnt_out,
              rowg, colg, pend_p, pend_c, gbuf, acc_v, cnt_v,
              sem_i, sem_g):
        c = lax.axis_index("c")
        s = lax.axis_index("s")

        one16f = jnp.ones((16,), jnp.float32)
        coff16 = jnp.full((16,), c * _N, jnp.int32)
        c32768 = jnp.full((16,), 32768, jnp.int32)
        c15 = jnp.full((16,), 15, jnp.int32)
        c32767 = jnp.full((16,), 32767, jnp.int32)

        for p in range(_PASSES):
            lo = p * _SPAN + s * _OWN
            lo16 = jnp.full((16,), lo, jnp.int32)
            hi16 = lo16 + _OWN
            pad_p16 = jnp.full(
                (16,), (lo + _OWN) * 32768 + 2 * _N, jnp.int32)

            pltpu.sync_copy(zacc_hbm, acc_v)
            pltpu.sync_copy(zcnt_hbm, cnt_v)

            ebase = c * (_NGRP * _GEDGES)
            pltpu.async_copy(r_hbm.at[pl.ds(ebase, _GEDGES)],
                             rowg.at[pl.ds(0, _GEDGES)], sem_i)
            pltpu.async_copy(c_hbm.at[pl.ds(ebase, _GEDGES)],
                             colg.at[pl.ds(0, _GEDGES)], sem_i)

            def _group(g, carry, lo16=lo16, hi16=hi16, pad_p16=pad_p16):
                slot = lax.rem(g, 2) * _GEDGES
                nxt = lax.rem(g + 1, 2) * _GEDGES
                sslot = lax.rem(g, 2)
                snxt = lax.rem(g + 1, 2)
                gn = jnp.minimum(g + 1, _NGRP - 1)
                pltpu.async_copy(
                    r_hbm.at[pl.ds(ebase + gn * _GEDGES, _GEDGES)],
                    rowg.at[pl.ds(nxt, _GEDGES)], sem_i)
                pltpu.async_copy(
                    c_hbm.at[pl.ds(ebase + gn * _GEDGES, _GEDGES)],
                    colg.at[pl.ds(nxt, _GEDGES)], sem_i)
                pltpu.make_async_copy(
                    r_hbm.at[pl.ds(ebase + g * _GEDGES, _GEDGES)],
                    rowg.at[pl.ds(slot, _GEDGES)], sem_i).wait()
                pltpu.make_async_copy(
                    c_hbm.at[pl.ds(ebase + g * _GEDGES, _GEDGES)],
                    colg.at[pl.ds(slot, _GEDGES)], sem_i).wait()

                def _scan(q, cur):
                    r16 = rowg[pl.ds(slot + q * 16, 16)]
                    c16 = colg[pl.ds(slot + q * 16, 16)]
                    m = (r16 >= lo16) & (r16 < hi16) & (r16 != c16)
                    packed = r16 * c32768 + (c16 + coff16)
                    _, sv, _ = plsc.sort_key_val(packed, packed, mask=m)
                    pend_p[pl.ds(cur, 16)] = sv
                    return cur + plsc.all_reduce_population_count(m)[0]
                cursor = lax.fori_loop(
                    0, _GEDGES // 16, _scan, jnp.int32(0))

                def _pad(t, carry2):
                    pend_p[pl.ds(cursor + t * 16, 16)] = pad_p16
                    return carry2
                lax.fori_loop(0, _BLK // 16, _pad, 0)
                nblk = jnp.maximum((cursor + _BLK - 1) // _BLK, 1)

                def _unpack(u, carry2):
                    v16 = pend_p[pl.ds(u * 16, 16)]
                    pend_c[pl.ds(u * 16, 16)] = v16 & c32767
                    return carry2
                lax.fori_loop(0, nblk * (_BLK // 16), _unpack, 0)

                pltpu.async_copy(
                    x_hbm.at[pend_c.at[pl.ds(0, _BLK)]],
                    gbuf.at[pl.ds(0, _BLK)], sem_g)

                def _drain(b, carry2):
                    bs = lax.rem(b, 2)
                    bn = lax.rem(b + 1, 2)
                    bnext = jnp.minimum(b + 1, nblk - 1)
                    pltpu.async_copy(
                        x_hbm.at[pend_c.at[pl.ds(bnext * _BLK, _BLK)]],
                        gbuf.at[pl.ds(bn * _BLK, _BLK)], sem_g)
                    pltpu.make_async_copy(
                        x_hbm.at[pend_c.at[pl.ds(b * _BLK, _BLK)]],
                        gbuf.at[pl.ds(bs * _BLK, _BLK)],
                        sem_g).wait()

                    def _sub(q2, carry3):
                        v16 = pend_p[pl.ds(b * _BLK + q2 * 16, 16)]
                        d16 = lax.shift_right_logical(v16, c15) - lo16
                        for e in range(16):
                            d = d16[e]
                            for k in range(_D // 16):
                                v = gbuf[bs * _BLK + q2 * 16 + e,
                                         pl.ds(16 * k, 16)]
                                plsc.addupdate(
                                    acc_v.at[d, pl.ds(16 * k, 16)], v)
                            plsc.addupdate(cnt_v.at[d, :], one16f)
                        return carry3
                    lax.fori_loop(0, _BLK // 16, _sub, 0)
                    return carry2
                lax.fori_loop(0, nblk, _drain, 0)
                pltpu.make_async_copy(
                    x_hbm.at[pend_c.at[pl.ds((nblk - 1) * _BLK, _BLK)]],
                    gbuf.at[pl.ds(lax.rem(nblk, 2) * _BLK, _BLK)],
                    sem_g).wait()
                return carry
            lax.fori_loop(0, _NGRP, _group, 0)
            pltpu.make_async_copy(
                r_hbm.at[pl.ds(ebase + (_NGRP - 1) * _GEDGES, _GEDGES)],
                rowg.at[pl.ds((_NGRP % 2) * _GEDGES, _GEDGES)],
                sem_i).wait()
            pltpu.make_async_copy(
                c_hbm.at[pl.ds(ebase + (_NGRP - 1) * _GEDGES, _GEDGES)],
                colg.at[pl.ds((_NGRP % 2) * _GEDGES, _GEDGES)],
                sem_i).wait()

            dst = c * _SIDE + lo
            pltpu.sync_copy(acc_v.at[pl.ds(0, _OWN)],
                            acc_out.at[pl.ds(dst, _OWN)])
            pltpu.sync_copy(cnt_v.at[pl.ds(0, _OWN)],
                            cnt_out.at[pl.ds(dst, _OWN)])

    zacc = jnp.zeros((_LACC, _D), jnp.float32)
    zcnt = jnp.zeros((_LACC, 16), jnp.float32)
    return sc_fn(x_cat, r_all, c_all, zacc, zcnt)


def _tc_epilogue(acc1, acc2, cnt1, cnt2, x1, x2, weight, bias2d):
    blk = 1000
    grid = (_N // blk,)

    def body(a1, a2, c1, c2, x1r, x2r, w, b, out):
        h1 = (a1[...] + x1r[...]) / (c1[...] + 1.0)
        h2 = (a2[...] + x2r[...]) / (c2[...] + 1.0)
        y = jnp.dot(h1, w[0:_D, :], preferred_element_type=jnp.float32)
        y += jnp.dot(h2, w[_D:2 * _D, :], preferred_element_type=jnp.float32)
        y += jnp.dot(x1r[...], w[2 * _D:3 * _D, :],
                     preferred_element_type=jnp.float32)
        y += b[...]
        nrm = jnp.sqrt(jnp.sum(y * y, axis=-1, keepdims=True))
        out[...] = y / jnp.maximum(nrm, 1e-12)

    row_spec = pl.BlockSpec((blk, _D), lambda i: (i, 0))
    cnt_spec = pl.BlockSpec((blk, 1), lambda i: (i, 0))
    return pl.pallas_call(
        body,
        grid=grid,
        in_specs=[
            row_spec, row_spec, cnt_spec, cnt_spec, row_spec, row_spec,
            pl.BlockSpec((3 * _D, _D), lambda i: (0, 0)),
            pl.BlockSpec((1, _D), lambda i: (0, 0)),
        ],
        out_specs=row_spec,
        out_shape=jax.ShapeDtypeStruct((_N, _D), jnp.float32),
    )(acc1, acc2, cnt1, cnt2, x1, x2, weight, bias2d)


def kernel(x_1, x_2, edge_index_pos, edge_index_neg, weight, bias):
    x_1 = x_1.astype(jnp.float32)
    x_2 = x_2.astype(jnp.float32)

    ep = edge_index_pos.astype(jnp.int32)
    en = edge_index_neg.astype(jnp.int32)
    zpad = jnp.zeros((_EPAD - _E,), jnp.int32)
    r_all = jnp.concatenate(
        [ep[0], zpad, en[0], zpad])
    c_all = jnp.concatenate(
        [ep[1], zpad, en[1], zpad])
    x_cat = jnp.concatenate(
        [x_1, x_2, jnp.zeros((16, _D), jnp.float32)], axis=0)

    acc, cnt = _sc_segment_sums(x_cat, r_all, c_all)

    acc1 = acc[0:_N]
    acc2 = acc[_SIDE:_SIDE + _N]
    cnt1 = cnt[0:_N, 0:1]
    cnt2 = cnt[_SIDE:_SIDE + _N, 0:1]

    return _tc_epilogue(acc1, acc2, cnt1, cnt2, x_1, x_2,
                        weight.astype(jnp.float32),
                        bias.astype(jnp.float32).reshape(1, _D))


# speedup vs baseline: 2.2969x; 2.2969x over previous
import functools
import jax
import jax.numpy as jnp
from jax import lax
from jax.experimental import pallas as pl
from jax.experimental.pallas import tpu as pltpu
from jax.experimental.pallas import tpu_sc as plsc

_N = 10000
_D = 128
_E = 320000
_TILES = 16
_PASSES = 3
_OWN = 240
_LACC = 256
_SPAN = _TILES * _OWN
_SIDE = _PASSES * _SPAN
_GEDGES = 7936
_EPAD = 325376
_NGRP = _EPAD // _GEDGES
_BLK = 128
_IBUF = _GEDGES + _BLK


def _sc_segment_sums(x_cat, r_all, c_all):
    mesh = plsc.VectorSubcoreMesh(core_axis_name="c", subcore_axis_name="s")

    @functools.partial(
        pl.kernel,
        out_type=[
            jax.ShapeDtypeStruct((2 * _SIDE, _D), jnp.float32),
            jax.ShapeDtypeStruct((2 * _SIDE, 16), jnp.float32),
        ],
        mesh=mesh,
        compiler_params=pltpu.CompilerParams(needs_layout_passes=False),
        scratch_types=[
            pltpu.VMEM((_IBUF,), jnp.int32),
            pltpu.VMEM((_IBUF,), jnp.int32),
            pltpu.VMEM((_BLK, _D), jnp.float32),
            pltpu.VMEM((_LACC, _D), jnp.float32),
            pltpu.VMEM((_LACC, 16), jnp.float32),
            pltpu.SemaphoreType.DMA,
            pltpu.SemaphoreType.DMA,
        ],
    )
    def sc_fn(x_hbm, r_hbm, c_hbm, zacc_hbm, zcnt_hbm,
              acc_out, cnt_out,
              rowg, colg, gbuf, acc_v, cnt_v, sem_i, sem_g):
        c = lax.axis_index("c")
        s = lax.axis_index("s")

        one16f = jnp.ones((16,), jnp.float32)
        coff16 = jnp.full((16,), c * _N, jnp.int32)
        c32768 = jnp.full((16,), 32768, jnp.int32)
        c15 = jnp.full((16,), 15, jnp.int32)
        c32767 = jnp.full((16,), 32767, jnp.int32)
        ebase = c * (_NGRP * _GEDGES)

        for p in range(_PASSES):
            lo = p * _SPAN + s * _OWN
            lo16 = jnp.full((16,), lo, jnp.int32)
            hi16 = lo16 + _OWN
            pad_p16 = jnp.full(
                (16,), (lo + _OWN) * 32768 + 2 * _N, jnp.int32)

            pltpu.sync_copy(zacc_hbm, acc_v)
            pltpu.sync_copy(zcnt_hbm, cnt_v)

            def _group(g, carry, lo16=lo16, hi16=hi16, pad_p16=pad_p16):
                h1 = pltpu.async_copy(
                    r_hbm.at[pl.ds(ebase + g * _GEDGES, _GEDGES)],
                    rowg.at[pl.ds(0, _GEDGES)], sem_i)
                h2 = pltpu.async_copy(
                    c_hbm.at[pl.ds(ebase + g * _GEDGES, _GEDGES)],
                    colg.at[pl.ds(0, _GEDGES)], sem_i)
                h1.wait()
                h2.wait()

                def _scan(q, cur):
                    r16 = rowg[pl.ds(q * 16, 16)]
                    c16 = colg[pl.ds(q * 16, 16)]
                    m = (r16 >= lo16) & (r16 < hi16) & (r16 != c16)
                    packed = r16 * c32768 + (c16 + coff16)
                    _, sv, _ = plsc.sort_key_val(packed, packed, mask=m)
                    rowg[pl.ds(cur, 16)] = sv
                    return cur + plsc.all_reduce_population_count(m)[0]
                cursor = lax.fori_loop(
                    0, _GEDGES // 16, _scan, jnp.int32(0))

                def _pad(t, carry2):
                    rowg[pl.ds(cursor + t * 16, 16)] = pad_p16
                    return carry2
                lax.fori_loop(0, _BLK // 16, _pad, 0)
                nblk = jnp.maximum((cursor + _BLK - 1) // _BLK, 1)

                def _unpack(u, carry2):
                    v16 = rowg[pl.ds(u * 16, 16)]
                    colg[pl.ds(u * 16, 16)] = v16 & c32767
                    return carry2
                lax.fori_loop(0, nblk * (_BLK // 16), _unpack, 0)

                def _drain(b, carry2):
                    pltpu.async_copy(
                        x_hbm.at[colg.at[pl.ds(b * _BLK, _BLK)]],
                        gbuf, sem_g).wait()

                    def _sub(q2, carry3):
                        v16 = rowg[pl.ds(b * _BLK + q2 * 16, 16)]
                        d16 = lax.shift_right_logical(v16, c15) - lo16
                        for e in range(16):
                            d = d16[e]
                            for k in range(_D // 16):
                                v = gbuf[q2 * 16 + e, pl.ds(16 * k, 16)]
                                plsc.addupdate(
                                    acc_v.at[d, pl.ds(16 * k, 16)], v)
                            plsc.addupdate(cnt_v.at[d, :], one16f)
                        return carry3
                    lax.fori_loop(0, _BLK // 16, _sub, 0)
                    return carry2
                lax.fori_loop(0, nblk, _drain, 0)
                return carry
            lax.fori_loop(0, _NGRP, _group, 0)

            dst = c * _SIDE + lo
            pltpu.sync_copy(acc_v.at[pl.ds(0, _OWN)],
                            acc_out.at[pl.ds(dst, _OWN)])
            pltpu.sync_copy(cnt_v.at[pl.ds(0, _OWN)],
                            cnt_out.at[pl.ds(dst, _OWN)])

    zacc = jnp.zeros((_LACC, _D), jnp.float32)
    zcnt = jnp.zeros((_LACC, 16), jnp.float32)
    return sc_fn(x_cat, r_all, c_all, zacc, zcnt)


def _tc_epilogue(acc1, acc2, cnt1, cnt2, x1, x2, weight, bias2d):
    blk = 1000
    grid = (_N // blk,)

    def body(a1, a2, c1, c2, x1r, x2r, w, b, out):
        h1 = (a1[...] + x1r[...]) / (c1[...] + 1.0)
        h2 = (a2[...] + x2r[...]) / (c2[...] + 1.0)
        y = jnp.dot(h1, w[0:_D, :], preferred_element_type=jnp.float32)
        y += jnp.dot(h2, w[_D:2 * _D, :], preferred_element_type=jnp.float32)
        y += jnp.dot(x1r[...], w[2 * _D:3 * _D, :],
                     preferred_element_type=jnp.float32)
        y += b[...]
        nrm = jnp.sqrt(jnp.sum(y * y, axis=-1, keepdims=True))
        out[...] = y / jnp.maximum(nrm, 1e-12)

    row_spec = pl.BlockSpec((blk, _D), lambda i: (i, 0))
    cnt_spec = pl.BlockSpec((blk, 1), lambda i: (i, 0))
    return pl.pallas_call(
        body,
        grid=grid,
        in_specs=[
            row_spec, row_spec, cnt_spec, cnt_spec, row_spec, row_spec,
            pl.BlockSpec((3 * _D, _D), lambda i: (0, 0)),
            pl.BlockSpec((1, _D), lambda i: (0, 0)),
        ],
        out_specs=row_spec,
        out_shape=jax.ShapeDtypeStruct((_N, _D), jnp.float32),
    )(acc1, acc2, cnt1, cnt2, x1, x2, weight, bias2d)


def kernel(x_1, x_2, edge_index_pos, edge_index_neg, weight, bias):
    x_1 = x_1.astype(jnp.float32)
    x_2 = x_2.astype(jnp.float32)

    ep = edge_index_pos.astype(jnp.int32)
    en = edge_index_neg.astype(jnp.int32)
    zpad = jnp.zeros((_EPAD - _E,), jnp.int32)
    r_all = jnp.concatenate(
        [ep[0], zpad, en[0], zpad])
    c_all = jnp.concatenate(
        [ep[1], zpad, en[1], zpad])
    x_cat = jnp.concatenate(
        [x_1, x_2, jnp.zeros((16, _D), jnp.float32)], axis=0)

    acc, cnt = _sc_segment_sums(x_cat, r_all, c_all)

    acc1 = acc[0:_N]
    acc2 = acc[_SIDE:_SIDE + _N]
    cnt1 = cnt[0:_N, 0:1]
    cnt2 = cnt[_SIDE:_SIDE + _N, 0:1]

    return _tc_epilogue(acc1, acc2, cnt1, cnt2, x_1, x_2,
                        weight.astype(jnp.float32),
                        bias.astype(jnp.float32).reshape(1, _D))


# speedup vs baseline: 3.0930x; 1.3466x over previous
import functools
import jax
import jax.numpy as jnp
from jax import lax
from jax.experimental import pallas as pl
from jax.experimental.pallas import tpu as pltpu
from jax.experimental.pallas import tpu_sc as plsc

_N = 10000
_D = 128
_E = 320000
_TILES = 16
_PASSES = 2
_OWN = 336
_LACC = 352
_SPAN = _TILES * _OWN
_SIDE = _PASSES * _SPAN
_GEDGES = 4096
_EPAD = 323584
_NGRP = _EPAD // _GEDGES
_BLK = 64
_IBUF = _GEDGES + _BLK


def _sc_segment_sums(x_cat, r_all, c_all):
    mesh = plsc.VectorSubcoreMesh(core_axis_name="c", subcore_axis_name="s")

    @functools.partial(
        pl.kernel,
        out_type=[
            jax.ShapeDtypeStruct((2 * _SIDE, _D), jnp.float32),
            jax.ShapeDtypeStruct((2 * _SIDE, 16), jnp.float32),
        ],
        mesh=mesh,
        compiler_params=pltpu.CompilerParams(needs_layout_passes=False),
        scratch_types=[
            pltpu.VMEM((_GEDGES,), jnp.int32),
            pltpu.VMEM((_IBUF,), jnp.int32),
            pltpu.VMEM((_IBUF,), jnp.int32),
            pltpu.VMEM((_BLK, _D), jnp.float32),
            pltpu.VMEM((_LACC, _D), jnp.float32),
            pltpu.VMEM((_LACC, 16), jnp.float32),
            pltpu.SemaphoreType.DMA,
            pltpu.SemaphoreType.DMA,
        ],
    )
    def sc_fn(x_hbm, r_hbm, c_hbm, zacc_hbm, zcnt_hbm,
              acc_out, cnt_out,
              rowg, colg, pend_p, gbuf, acc_v, cnt_v, sem_i, sem_g):
        c = lax.axis_index("c")
        s = lax.axis_index("s")

        one16f = jnp.ones((16,), jnp.float32)
        coff16 = jnp.full((16,), c * _N, jnp.int32)
        c32768 = jnp.full((16,), 32768, jnp.int32)
        c15 = jnp.full((16,), 15, jnp.int32)
        c32767 = jnp.full((16,), 32767, jnp.int32)
        ebase = c * (_NGRP * _GEDGES)

        for p in range(_PASSES):
            lo = p * _SPAN + s * _OWN
            lo16 = jnp.full((16,), lo, jnp.int32)
            hi16 = lo16 + _OWN
            pad_p16 = jnp.full(
                (16,), (lo + _OWN) * 32768 + 2 * _N, jnp.int32)

            pltpu.sync_copy(zacc_hbm, acc_v)
            pltpu.sync_copy(zcnt_hbm, cnt_v)

            def _group(g, carry, lo16=lo16, hi16=hi16, pad_p16=pad_p16):
                h1 = pltpu.async_copy(
                    r_hbm.at[pl.ds(ebase + g * _GEDGES, _GEDGES)],
                    rowg.at[pl.ds(0, _GEDGES)], sem_i)
                h2 = pltpu.async_copy(
                    c_hbm.at[pl.ds(ebase + g * _GEDGES, _GEDGES)],
                    colg.at[pl.ds(0, _GEDGES)], sem_i)
                h1.wait()
                h2.wait()

                def _scan(q, cur):
                    r16 = rowg[pl.ds(q * 16, 16)]
                    c16 = colg[pl.ds(q * 16, 16)]
                    m = (r16 >= lo16) & (r16 < hi16) & (r16 != c16)
                    cnt = plsc.all_reduce_population_count(m)[0]

                    @pl.when(cnt > 0)
                    def _():
                        packed = r16 * c32768 + (c16 + coff16)
                        _, sv, _ = plsc.sort_key_val(packed, packed, mask=m)
                        pend_p[pl.ds(cur, 16)] = sv
                    return cur + cnt
                cursor = lax.fori_loop(
                    0, _GEDGES // 16, _scan, jnp.int32(0))

                def _pad(t, carry2):
                    pend_p[pl.ds(cursor + t * 16, 16)] = pad_p16
                    return carry2
                lax.fori_loop(0, _BLK // 16, _pad, 0)
                nblk = jnp.maximum((cursor + _BLK - 1) // _BLK, 1)

                def _unpack(u, carry2):
                    v16 = pend_p[pl.ds(u * 16, 16)]
                    colg[pl.ds(u * 16, 16)] = v16 & c32767
                    return carry2
                lax.fori_loop(0, nblk * (_BLK // 16), _unpack, 0)

                def _drain(b, carry2):
                    pltpu.async_copy(
                        x_hbm.at[colg.at[pl.ds(b * _BLK, _BLK)]],
                        gbuf, sem_g).wait()

                    def _sub(q2, carry3):
                        v16 = pend_p[pl.ds(b * _BLK + q2 * 16, 16)]
                        d16 = lax.shift_right_logical(v16, c15) - lo16
                        for e in range(16):
                            d = d16[e]
                            for k in range(_D // 16):
                                v = gbuf[q2 * 16 + e, pl.ds(16 * k, 16)]
                                plsc.addupdate(
                                    acc_v.at[d, pl.ds(16 * k, 16)], v)
                            plsc.addupdate(cnt_v.at[d, :], one16f)
                        return carry3
                    lax.fori_loop(0, _BLK // 16, _sub, 0)
                    return carry2
                lax.fori_loop(0, nblk, _drain, 0)
                return carry
            lax.fori_loop(0, _NGRP, _group, 0)

            dst = c * _SIDE + lo
            pltpu.sync_copy(acc_v.at[pl.ds(0, _OWN)],
                            acc_out.at[pl.ds(dst, _OWN)])
            pltpu.sync_copy(cnt_v.at[pl.ds(0, _OWN)],
                            cnt_out.at[pl.ds(dst, _OWN)])

    zacc = jnp.zeros((_LACC, _D), jnp.float32)
    zcnt = jnp.zeros((_LACC, 16), jnp.float32)
    return sc_fn(x_cat, r_all, c_all, zacc, zcnt)


def _tc_epilogue(acc1, acc2, cnt1, cnt2, x1, x2, weight, bias2d):
    blk = 1000
    grid = (_N // blk,)

    def body(a1, a2, c1, c2, x1r, x2r, w, b, out):
        h1 = (a1[...] + x1r[...]) / (c1[...] + 1.0)
        h2 = (a2[...] + x2r[...]) / (c2[...] + 1.0)
        y = jnp.dot(h1, w[0:_D, :], preferred_element_type=jnp.float32)
        y += jnp.dot(h2, w[_D:2 * _D, :], preferred_element_type=jnp.float32)
        y += jnp.dot(x1r[...], w[2 * _D:3 * _D, :],
                     preferred_element_type=jnp.float32)
        y += b[...]
        nrm = jnp.sqrt(jnp.sum(y * y, axis=-1, keepdims=True))
        out[...] = y / jnp.maximum(nrm, 1e-12)

    row_spec = pl.BlockSpec((blk, _D), lambda i: (i, 0))
    cnt_spec = pl.BlockSpec((blk, 1), lambda i: (i, 0))
    return pl.pallas_call(
        body,
        grid=grid,
        in_specs=[
            row_spec, row_spec, cnt_spec, cnt_spec, row_spec, row_spec,
            pl.BlockSpec((3 * _D, _D), lambda i: (0, 0)),
            pl.BlockSpec((1, _D), lambda i: (0, 0)),
        ],
        out_specs=row_spec,
        out_shape=jax.ShapeDtypeStruct((_N, _D), jnp.float32),
    )(acc1, acc2, cnt1, cnt2, x1, x2, weight, bias2d)


def kernel(x_1, x_2, edge_index_pos, edge_index_neg, weight, bias):
    x_1 = x_1.astype(jnp.float32)
    x_2 = x_2.astype(jnp.float32)

    ep = edge_index_pos.astype(jnp.int32)
    en = edge_index_neg.astype(jnp.int32)
    zpad = jnp.zeros((_EPAD - _E,), jnp.int32)
    r_all = jnp.concatenate(
        [ep[0], zpad, en[0], zpad])
    c_all = jnp.concatenate(
        [ep[1], zpad, en[1], zpad])
    x_cat = jnp.concatenate(
        [x_1, x_2, jnp.zeros((16, _D), jnp.float32)], axis=0)

    acc, cnt = _sc_segment_sums(x_cat, r_all, c_all)

    acc1 = acc[0:_N]
    acc2 = acc[_SIDE:_SIDE + _N]
    cnt1 = cnt[0:_N, 0:1]
    cnt2 = cnt[_SIDE:_SIDE + _N, 0:1]

    return _tc_epilogue(acc1, acc2, cnt1, cnt2, x_1, x_2,
                        weight.astype(jnp.float32),
                        bias.astype(jnp.float32).reshape(1, _D))


# speedup vs baseline: 3.1015x; 1.0028x over previous
import functools
import jax
import jax.numpy as jnp
from jax import lax
from jax.experimental import pallas as pl
from jax.experimental.pallas import tpu as pltpu
from jax.experimental.pallas import tpu_sc as plsc

_N = 10000
_D = 128
_E = 320000
_TILES = 16
_PASSES = 2
_OWN = 336
_LACC = 352
_SPAN = _TILES * _OWN
_SIDE = _PASSES * _SPAN
_GEDGES = 4096
_EPAD = 323584
_NGRP = _EPAD // _GEDGES
_BLK = 64
_IBUF = _GEDGES + _BLK


def _sc_segment_sums(x_cat, e_all):
    mesh = plsc.VectorSubcoreMesh(core_axis_name="c", subcore_axis_name="s")

    @functools.partial(
        pl.kernel,
        out_type=[
            jax.ShapeDtypeStruct((2 * _SIDE, _D), jnp.float32),
            jax.ShapeDtypeStruct((2 * _SIDE, 16), jnp.float32),
        ],
        mesh=mesh,
        compiler_params=pltpu.CompilerParams(needs_layout_passes=False),
        scratch_types=[
            pltpu.VMEM((_GEDGES,), jnp.int32),
            pltpu.VMEM((_IBUF,), jnp.int32),
            pltpu.VMEM((_IBUF,), jnp.int32),
            pltpu.VMEM((_BLK, _D), jnp.float32),
            pltpu.VMEM((_LACC, _D), jnp.float32),
            pltpu.VMEM((_LACC, 16), jnp.float32),
            pltpu.SemaphoreType.DMA,
            pltpu.SemaphoreType.DMA,
        ],
    )
    def sc_fn(x_hbm, e_hbm, zacc_hbm, zcnt_hbm,
              acc_out, cnt_out,
              eg, colg, pend_p, gbuf, acc_v, cnt_v, sem_i, sem_g):
        c = lax.axis_index("c")
        s = lax.axis_index("s")

        one16f = jnp.ones((16,), jnp.float32)
        coff16 = jnp.full((16,), c * _N, jnp.int32)
        c32768 = jnp.full((16,), 32768, jnp.int32)
        c15 = jnp.full((16,), 15, jnp.int32)
        c32767 = jnp.full((16,), 32767, jnp.int32)
        ebase = c * (_NGRP * _GEDGES)

        for p in range(_PASSES):
            lo = p * _SPAN + s * _OWN
            lo16 = jnp.full((16,), lo, jnp.int32)
            hi16 = lo16 + _OWN
            pad_p16 = jnp.full(
                (16,), (lo + _OWN) * 32768 + 2 * _N, jnp.int32) - coff16

            pltpu.sync_copy(zacc_hbm, acc_v)
            pltpu.sync_copy(zcnt_hbm, cnt_v)

            def _group(g, carry, lo16=lo16, hi16=hi16, pad_p16=pad_p16):
                pltpu.async_copy(
                    e_hbm.at[pl.ds(ebase + g * _GEDGES, _GEDGES)],
                    eg, sem_i).wait()

                def _scan(q, cur):
                    v = eg[pl.ds(q * 16, 16)]
                    r16 = lax.shift_right_logical(v, c15)
                    c16 = v & c32767
                    m = (r16 >= lo16) & (r16 < hi16) & (r16 != c16)
                    cnt = plsc.all_reduce_population_count(m)[0]

                    @pl.when(cnt > 0)
                    def _():
                        _, sv, _ = plsc.sort_key_val(v, v, mask=m)
                        pend_p[pl.ds(cur, 16)] = sv
                    return cur + cnt
                cursor = lax.fori_loop(
                    0, _GEDGES // 16, _scan, jnp.int32(0))

                def _pad(t, carry2):
                    pend_p[pl.ds(cursor + t * 16, 16)] = pad_p16
                    return carry2
                lax.fori_loop(0, _BLK // 16, _pad, 0)
                nblk = jnp.maximum((cursor + _BLK - 1) // _BLK, 1)

                def _unpack(u, carry2):
                    v16 = pend_p[pl.ds(u * 16, 16)]
                    colg[pl.ds(u * 16, 16)] = (v16 & c32767) + coff16
                    return carry2
                lax.fori_loop(0, nblk * (_BLK // 16), _unpack, 0)

                def _drain(b, carry2):
                    pltpu.async_copy(
                        x_hbm.at[colg.at[pl.ds(b * _BLK, _BLK)]],
                        gbuf, sem_g).wait()

                    def _sub(q2, carry3):
                        v16 = pend_p[pl.ds(b * _BLK + q2 * 16, 16)]
                        d16 = lax.shift_right_logical(v16, c15) - lo16
                        for e in range(16):
                            d = d16[e]
                            for k in range(_D // 16):
                                v = gbuf[q2 * 16 + e, pl.ds(16 * k, 16)]
                                plsc.addupdate(
                                    acc_v.at[d, pl.ds(16 * k, 16)], v)
                            plsc.addupdate(cnt_v.at[d, :], one16f)
                        return carry3
                    lax.fori_loop(0, _BLK // 16, _sub, 0)
                    return carry2
                lax.fori_loop(0, nblk, _drain, 0)
                return carry
            lax.fori_loop(0, _NGRP, _group, 0)

            dst = c * _SIDE + lo
            pltpu.sync_copy(acc_v.at[pl.ds(0, _OWN)],
                            acc_out.at[pl.ds(dst, _OWN)])
            pltpu.sync_copy(cnt_v.at[pl.ds(0, _OWN)],
                            cnt_out.at[pl.ds(dst, _OWN)])

    zacc = jnp.zeros((_LACC, _D), jnp.float32)
    zcnt = jnp.zeros((_LACC, 16), jnp.float32)
    return sc_fn(x_cat, e_all, zacc, zcnt)


def _tc_epilogue(acc1, acc2, cnt1, cnt2, x1, x2, weight, bias2d):
    blk = 1000
    grid = (_N // blk,)

    def body(a1, a2, c1, c2, x1r, x2r, w, b, out):
        h1 = (a1[...] + x1r[...]) / (c1[...] + 1.0)
        h2 = (a2[...] + x2r[...]) / (c2[...] + 1.0)
        y = jnp.dot(h1, w[0:_D, :], preferred_element_type=jnp.float32)
        y += jnp.dot(h2, w[_D:2 * _D, :], preferred_element_type=jnp.float32)
        y += jnp.dot(x1r[...], w[2 * _D:3 * _D, :],
                     preferred_element_type=jnp.float32)
        y += b[...]
        nrm = jnp.sqrt(jnp.sum(y * y, axis=-1, keepdims=True))
        out[...] = y / jnp.maximum(nrm, 1e-12)

    row_spec = pl.BlockSpec((blk, _D), lambda i: (i, 0))
    cnt_spec = pl.BlockSpec((blk, 1), lambda i: (i, 0))
    return pl.pallas_call(
        body,
        grid=grid,
        in_specs=[
            row_spec, row_spec, cnt_spec, cnt_spec, row_spec, row_spec,
            pl.BlockSpec((3 * _D, _D), lambda i: (0, 0)),
            pl.BlockSpec((1, _D), lambda i: (0, 0)),
        ],
        out_specs=row_spec,
        out_shape=jax.ShapeDtypeStruct((_N, _D), jnp.float32),
    )(acc1, acc2, cnt1, cnt2, x1, x2, weight, bias2d)


def kernel(x_1, x_2, edge_index_pos, edge_index_neg, weight, bias):
    x_1 = x_1.astype(jnp.float32)
    x_2 = x_2.astype(jnp.float32)

    ep = edge_index_pos.astype(jnp.int32)
    en = edge_index_neg.astype(jnp.int32)
    zpad = jnp.zeros((_EPAD - _E,), jnp.int32)
    e_all = jnp.concatenate(
        [ep[0] * 32768 + ep[1], zpad, en[0] * 32768 + en[1], zpad])
    x_cat = jnp.concatenate(
        [x_1, x_2, jnp.zeros((16, _D), jnp.float32)], axis=0)

    acc, cnt = _sc_segment_sums(x_cat, e_all)

    acc1 = acc[0:_N]
    acc2 = acc[_SIDE:_SIDE + _N]
    cnt1 = cnt[0:_N, 0:1]
    cnt2 = cnt[_SIDE:_SIDE + _N, 0:1]

    return _tc_epilogue(acc1, acc2, cnt1, cnt2, x_1, x_2,
                        weight.astype(jnp.float32),
                        bias.astype(jnp.float32).reshape(1, _D))
